# baseline (speedup 1.0000x reference)
import functools

import jax
import jax.numpy as jnp
from jax import lax
from jax.experimental import pallas as pl
from jax.experimental.pallas import tpu as pltpu
from jax.experimental.pallas import tpu_sc as plsc

B = 2
T = 2048
D = 2048
P = 4096
EPS = 1e-8

LANES = 16
NC, NS = 2, 16
NW = NC * NS
PPW = P // NW
CH = 8
NCHUNK = PPW // CH
TI = T // 2
NSTEP = TI // LANES
UNR = 2


def _softplus(x):
    return jnp.maximum(x, 0.0) + jnp.log1p(jnp.exp(-jnp.abs(x)))


def _weights_body(r_ref, le_ref, lo_ref, qw_ref):
    r = _softplus(r_ref[:, :])
    li = lax.broadcasted_iota(jnp.int32, (P, LANES), 1).astype(jnp.float32)
    geo = (jnp.exp(-r * T) - 1.0) / (jnp.exp(-r) - 1.0)
    invden = lax.rsqrt(geo + EPS)
    le_ref[:, :] = jnp.exp(-r * (31.0 - 2.0 * li)) * invden
    lo_ref[:, :] = jnp.exp(-r * (30.0 - 2.0 * li)) * invden
    qw_ref[:, :] = jnp.broadcast_to(jnp.exp(-32.0 * r), (P, LANES))


_weights_tc = pl.pallas_call(
    _weights_body,
    out_shape=[
        jax.ShapeDtypeStruct((P, LANES), jnp.float32),
        jax.ShapeDtypeStruct((P, LANES), jnp.float32),
        jax.ShapeDtypeStruct((P, LANES), jnp.float32),
    ],
)


def _finalize_body(part_ref, out_ref):
    out_ref[:, :] = jnp.sum(part_ref[:, :, :], axis=2)


_finalize_tc = pl.pallas_call(
    _finalize_body,
    out_shape=jax.ShapeDtypeStruct((B, P), jnp.float32),
)


@functools.partial(
    pl.kernel,
    mesh=plsc.VectorSubcoreMesh(core_axis_name="c", subcore_axis_name="s"),
    out_type=jax.ShapeDtypeStruct((B, P * LANES), jnp.float32),
    scratch_types=[
        pltpu.VMEM((4 * PPW,), jnp.int32),
        pltpu.VMEM((PPW * LANES,), jnp.float32),
        pltpu.VMEM((PPW * LANES,), jnp.float32),
        pltpu.VMEM((PPW * LANES,), jnp.float32),
        pltpu.VMEM((4 * CH, TI), jnp.int32),
        pltpu.VMEM((4 * CH, TI), jnp.int32),
        pltpu.VMEM((B * PPW * LANES,), jnp.float32),
        pltpu.SemaphoreType.DMA,
        pltpu.SemaphoreType.DMA,
    ],
)
def _sync_sc(zT_hbm, gidx_hbm, le_hbm, lo_hbm, qw_hbm, out_hbm,
             idx_v, le_v, lo_v, qw_v, rows0_v, rows1_v, part_v, sem0, sem1):
    wid = lax.axis_index("s") * NC + lax.axis_index("c")
    pbase = wid * PPW
    pltpu.sync_copy(gidx_hbm.at[pl.ds(pbase * 4, PPW * 4)], idx_v)
    pltpu.sync_copy(le_hbm.at[pl.ds(pbase * LANES, PPW * LANES)], le_v)
    pltpu.sync_copy(lo_hbm.at[pl.ds(pbase * LANES, PPW * LANES)], lo_v)
    pltpu.sync_copy(qw_hbm.at[pl.ds(pbase * LANES, PPW * LANES)], qw_v)

    def gather(ch, buf, sem):
        return pltpu.make_async_copy(
            zT_hbm.at[idx_v.at[pl.ds(ch * 4 * CH, 4 * CH)]], buf, sem
        )

    himask = jnp.full((LANES,), -65536, jnp.int32)
    sh16 = jnp.full((LANES,), 16, jnp.int32)

    def ext(x):
        e = lax.bitcast_convert_type(lax.shift_left(x, sh16), jnp.float32)
        o = lax.bitcast_convert_type(x & himask, jnp.float32)
        return e, o

    def wprod(xi, xj, levec, lovec):
        zie, zio = ext(xi)
        zje, zjo = ext(xj)
        return (zie * zje) * levec + (zio * zjo) * lovec

    def compute(ch, buf):
        def pair_body(k, carry2):
            pk = ch * CH + k
            levec = le_v[pl.ds(pk * LANES, LANES)]
            lovec = lo_v[pl.ds(pk * LANES, LANES)]
            qvec = qw_v[pl.ds(pk * LANES, LANES)]
            q2 = qvec * qvec

            def t_body(m, accs):
                accs = list(accs)
                base = m * (UNR * LANES)
                for u in range(UNR):
                    s = pl.ds(base + u * LANES, LANES)
                    accs[u] = accs[u] * q2 + wprod(
                        buf[4 * k + 0, s], buf[4 * k + 1, s], levec, lovec)
                    accs[UNR + u] = accs[UNR + u] * q2 + wprod(
                        buf[4 * k + 2, s], buf[4 * k + 3, s], levec, lovec)
                return tuple(accs)

            z16 = jnp.zeros((LANES,), jnp.float32)
            accs = lax.fori_loop(0, NSTEP // UNR, t_body, (z16,) * (2 * UNR))
            a0 = accs[0] * qvec + accs[1]
            a1 = accs[2] * qvec + accs[3]
            part_v[pl.ds(pk * LANES, LANES)] = a0
            part_v[pl.ds((PPW + pk) * LANES, LANES)] = a1
            return carry2

        lax.fori_loop(0, CH, pair_body, 0)

    gather(0, rows0_v, sem0).start()
    gather(1, rows1_v, sem1).start()

    def g_body(g, carry):
        ch0 = 2 * g
        ch1 = 2 * g + 1
        gather(ch0, rows0_v, sem0).wait()
        compute(ch0, rows0_v)

        @pl.when(ch0 + 2 < NCHUNK)
        def _():
            gather(ch0 + 2, rows0_v, sem0).start()

        gather(ch1, rows1_v, sem1).wait()
        compute(ch1, rows1_v)

        @pl.when(ch1 + 2 < NCHUNK)
        def _():
            gather(ch1 + 2, rows1_v, sem1).start()

        return carry

    lax.fori_loop(0, NCHUNK // 2, g_body, 0)

    pltpu.sync_copy(
        part_v.at[pl.ds(0, PPW * LANES)],
        out_hbm.at[0, pl.ds(pbase * LANES, PPW * LANES)],
    )
    pltpu.sync_copy(
        part_v.at[pl.ds(PPW * LANES, PPW * LANES)],
        out_hbm.at[1, pl.ds(pbase * LANES, PPW * LANES)],
    )


def kernel(z_history, decay_rates, indices_i, indices_j):
    zT = jnp.swapaxes(z_history.astype(jnp.bfloat16), 1, 2).reshape(B * D, T)
    zTi = lax.bitcast_convert_type(zT.reshape(B * D, TI, 2), jnp.int32)
    gidx = jnp.stack(
        [indices_i, indices_j, indices_i + D, indices_j + D], axis=1
    ).reshape(-1).astype(jnp.int32)
    le, lo, qw = _weights_tc(decay_rates.reshape(P, 1))
    le = le.reshape(P * LANES)
    lo = lo.reshape(P * LANES)
    qw = qw.reshape(P * LANES)
    partial = _sync_sc(zTi, gidx, le, lo, qw)
    return _finalize_tc(partial.reshape(B, P, LANES))


# speedup vs baseline: 2.1694x; 2.1694x over previous
import functools

import jax
import jax.numpy as jnp
from jax import lax
from jax.experimental import pallas as pl
from jax.experimental.pallas import tpu as pltpu
from jax.experimental.pallas import tpu_sc as plsc

B = 2
T = 2048
D = 2048
P = 4096
EPS = 1e-8

LANES = 16
NC, NS = 2, 16
NW = NC * NS
PPW = P // NW
CH = 4
NCHUNK = PPW // CH
TCH = T // LANES
UNR = 4


def _softplus(x):
    return jnp.maximum(x, 0.0) + jnp.log1p(jnp.exp(-jnp.abs(x)))


def _weights_body(r_ref, lw_ref, qw_ref):
    r = _softplus(r_ref[:, :])
    li = lax.broadcasted_iota(jnp.int32, (P, LANES), 1).astype(jnp.float32)
    geo = (jnp.exp(-r * T) - 1.0) / (jnp.exp(-r) - 1.0)
    invden = lax.rsqrt(geo + EPS)
    lw_ref[:, :] = jnp.exp(-r * (15.0 - li)) * invden
    qw_ref[:, :] = jnp.broadcast_to(jnp.exp(-16.0 * r), (P, LANES))


_weights_tc = pl.pallas_call(
    _weights_body,
    out_shape=[
        jax.ShapeDtypeStruct((P, LANES), jnp.float32),
        jax.ShapeDtypeStruct((P, LANES), jnp.float32),
    ],
)


def _finalize_body(part_ref, out_ref):
    out_ref[:, :] = jnp.sum(part_ref[:, :, :], axis=2)


_finalize_tc = pl.pallas_call(
    _finalize_body,
    out_shape=jax.ShapeDtypeStruct((B, P), jnp.float32),
)


@functools.partial(
    pl.kernel,
    mesh=plsc.VectorSubcoreMesh(core_axis_name="c", subcore_axis_name="s"),
    out_type=jax.ShapeDtypeStruct((B, P * LANES), jnp.float32),
    scratch_types=[
        pltpu.VMEM((4 * PPW,), jnp.int32),
        pltpu.VMEM((PPW * LANES,), jnp.float32),
        pltpu.VMEM((PPW * LANES,), jnp.float32),
        pltpu.VMEM((4 * CH, T), jnp.float32),
        pltpu.VMEM((4 * CH, T), jnp.float32),
        pltpu.VMEM((B * PPW * LANES,), jnp.float32),
        pltpu.SemaphoreType.DMA,
        pltpu.SemaphoreType.DMA,
    ],
)
def _sync_sc(zT_hbm, gidx_hbm, lw_hbm, qw_hbm, out_hbm,
             idx_v, lw_v, qw_v, rows0_v, rows1_v, part_v, sem0, sem1):
    wid = lax.axis_index("s") * NC + lax.axis_index("c")
    pbase = wid * PPW
    pltpu.sync_copy(gidx_hbm.at[pl.ds(pbase * 4, PPW * 4)], idx_v)
    pltpu.sync_copy(lw_hbm.at[pl.ds(pbase * LANES, PPW * LANES)], lw_v)
    pltpu.sync_copy(qw_hbm.at[pl.ds(pbase * LANES, PPW * LANES)], qw_v)

    def gather(ch, buf, sem):
        return pltpu.make_async_copy(
            zT_hbm.at[idx_v.at[pl.ds(ch * 4 * CH, 4 * CH)]], buf, sem
        )

    def compute(ch, buf):
        def pair_body(k, carry2):
            pk = ch * CH + k
            lvec = lw_v[pl.ds(pk * LANES, LANES)]
            qvec = qw_v[pl.ds(pk * LANES, LANES)]
            q2 = qvec * qvec
            q4 = q2 * q2

            def t_body(m, accs):
                accs = list(accs)
                base = m * (UNR * LANES)
                for u in range(UNR):
                    s = pl.ds(base + u * LANES, LANES)
                    p0 = buf[4 * k + 0, s] * buf[4 * k + 1, s]
                    p1 = buf[4 * k + 2, s] * buf[4 * k + 3, s]
                    accs[u] = accs[u] * q4 + p0 * lvec
                    accs[UNR + u] = accs[UNR + u] * q4 + p1 * lvec
                return tuple(accs)

            z16 = jnp.zeros((LANES,), jnp.float32)
            accs = lax.fori_loop(0, TCH // UNR, t_body, (z16,) * (2 * UNR))
            a0 = ((accs[0] * qvec + accs[1]) * qvec + accs[2]) * qvec + accs[3]
            a1 = ((accs[4] * qvec + accs[5]) * qvec + accs[6]) * qvec + accs[7]
            part_v[pl.ds(pk * LANES, LANES)] = a0
            part_v[pl.ds((PPW + pk) * LANES, LANES)] = a1
            return carry2

        lax.fori_loop(0, CH, pair_body, 0)

    gather(0, rows0_v, sem0).start()
    gather(1, rows1_v, sem1).start()

    def g_body(g, carry):
        ch0 = 2 * g
        ch1 = 2 * g + 1
        gather(ch0, rows0_v, sem0).wait()
        compute(ch0, rows0_v)

        @pl.when(ch0 + 2 < NCHUNK)
        def _():
            gather(ch0 + 2, rows0_v, sem0).start()

        gather(ch1, rows1_v, sem1).wait()
        compute(ch1, rows1_v)

        @pl.when(ch1 + 2 < NCHUNK)
        def _():
            gather(ch1 + 2, rows1_v, sem1).start()

        return carry

    lax.fori_loop(0, NCHUNK // 2, g_body, 0)

    pltpu.sync_copy(
        part_v.at[pl.ds(0, PPW * LANES)],
        out_hbm.at[0, pl.ds(pbase * LANES, PPW * LANES)],
    )
    pltpu.sync_copy(
        part_v.at[pl.ds(PPW * LANES, PPW * LANES)],
        out_hbm.at[1, pl.ds(pbase * LANES, PPW * LANES)],
    )


def kernel(z_history, decay_rates, indices_i, indices_j):
    zT = jnp.swapaxes(z_history, 1, 2).reshape(B * D, T)
    gidx = jnp.stack(
        [indices_i, indices_j, indices_i + D, indices_j + D], axis=1
    ).reshape(-1).astype(jnp.int32)
    lw, qw = _weights_tc(decay_rates.reshape(P, 1))
    lw = lw.reshape(P * LANES)
    qw = qw.reshape(P * LANES)
    partial = _sync_sc(zT, gidx, lw, qw)
    return _finalize_tc(partial.reshape(B, P, LANES))


# speedup vs baseline: 2.3418x; 1.0795x over previous
import functools

import jax
import jax.numpy as jnp
from jax import lax
from jax.experimental import pallas as pl
from jax.experimental.pallas import tpu as pltpu
from jax.experimental.pallas import tpu_sc as plsc

B = 2
T = 2048
D = 2048
P = 4096
EPS = 1e-8

LANES = 16
NC, NS = 2, 16
NW = NC * NS
PPW = P // NW
CH = 4
NCHUNK = PPW // CH
TCH = T // LANES
UNR = 4


def _softplus(x):
    return jnp.maximum(x, 0.0) + jnp.log1p(jnp.exp(-jnp.abs(x)))


def _weights_body(r_ref, lw_ref, qw_ref):
    r = _softplus(r_ref[:, :])
    li = lax.broadcasted_iota(jnp.int32, (P, LANES), 1).astype(jnp.float32)
    geo = (jnp.exp(-r * T) - 1.0) / (jnp.exp(-r) - 1.0)
    invden = lax.rsqrt(geo + EPS)
    lw_ref[:, :] = jnp.exp(-r * (15.0 - li)) * invden
    qw_ref[:, :] = jnp.broadcast_to(jnp.exp(-16.0 * r), (P, LANES))


_weights_tc = pl.pallas_call(
    _weights_body,
    out_shape=[
        jax.ShapeDtypeStruct((P, LANES), jnp.float32),
        jax.ShapeDtypeStruct((P, LANES), jnp.float32),
    ],
)


@functools.partial(
    pl.kernel,
    mesh=plsc.VectorSubcoreMesh(core_axis_name="c", subcore_axis_name="s"),
    compiler_params=pltpu.CompilerParams(needs_layout_passes=False),
    out_type=jax.ShapeDtypeStruct((B, P), jnp.float32),
    scratch_types=[
        pltpu.VMEM((4 * PPW,), jnp.int32),
        pltpu.VMEM((PPW * LANES,), jnp.float32),
        pltpu.VMEM((PPW * LANES,), jnp.float32),
        pltpu.VMEM((4 * CH, T), jnp.float32),
        pltpu.VMEM((4 * CH, T), jnp.float32),
        pltpu.VMEM((B * PPW * LANES,), jnp.float32),
        pltpu.VMEM((B * PPW,), jnp.float32),
        pltpu.SemaphoreType.DMA,
        pltpu.SemaphoreType.DMA,
    ],
)
def _sync_sc(zT_hbm, gidx_hbm, lw_hbm, qw_hbm, out_hbm,
             idx_v, lw_v, qw_v, rows0_v, rows1_v, part_v, red_v, sem0, sem1):
    wid = lax.axis_index("s") * NC + lax.axis_index("c")
    pbase = wid * PPW
    pltpu.sync_copy(gidx_hbm.at[pl.ds(pbase * 4, PPW * 4)], idx_v)
    pltpu.sync_copy(lw_hbm.at[pl.ds(pbase * LANES, PPW * LANES)], lw_v)
    pltpu.sync_copy(qw_hbm.at[pl.ds(pbase * LANES, PPW * LANES)], qw_v)

    def gather(ch, buf, sem):
        return pltpu.make_async_copy(
            zT_hbm.at[idx_v.at[pl.ds(ch * 4 * CH, 4 * CH)]], buf, sem
        )

    def compute(ch, buf):
        def pair_body(k, carry2):
            pk = ch * CH + k
            lvec = lw_v[pl.ds(pk * LANES, LANES)]
            qvec = qw_v[pl.ds(pk * LANES, LANES)]
            q2 = qvec * qvec
            q4 = q2 * q2

            def t_body(m, accs):
                accs = list(accs)
                base = m * (UNR * LANES)
                for u in range(UNR):
                    s = pl.ds(base + u * LANES, LANES)
                    p0 = buf[4 * k + 0, s] * buf[4 * k + 1, s]
                    p1 = buf[4 * k + 2, s] * buf[4 * k + 3, s]
                    accs[u] = accs[u] * q4 + p0 * lvec
                    accs[UNR + u] = accs[UNR + u] * q4 + p1 * lvec
                return tuple(accs)

            z16 = jnp.zeros((LANES,), jnp.float32)
            accs = lax.fori_loop(0, TCH // UNR, t_body, (z16,) * (2 * UNR))
            a0 = ((accs[0] * qvec + accs[1]) * qvec + accs[2]) * qvec + accs[3]
            a1 = ((accs[4] * qvec + accs[5]) * qvec + accs[6]) * qvec + accs[7]
            part_v[pl.ds(pk * LANES, LANES)] = a0
            part_v[pl.ds((PPW + pk) * LANES, LANES)] = a1
            return carry2

        lax.fori_loop(0, CH, pair_body, 0)

    gather(0, rows0_v, sem0).start()
    gather(1, rows1_v, sem1).start()

    def g_body(g, carry):
        ch0 = 2 * g
        ch1 = 2 * g + 1
        gather(ch0, rows0_v, sem0).wait()
        compute(ch0, rows0_v)

        @pl.when(ch0 + 2 < NCHUNK)
        def _():
            gather(ch0 + 2, rows0_v, sem0).start()

        gather(ch1, rows1_v, sem1).wait()
        compute(ch1, rows1_v)

        @pl.when(ch1 + 2 < NCHUNK)
        def _():
            gather(ch1 + 2, rows1_v, sem1).start()

        return carry

    lax.fori_loop(0, NCHUNK // 2, g_body, 0)

    lane = lax.iota(jnp.int32, LANES)

    def red_body(g, carry):
        idx0 = (g * LANES + lane) * LANES
        acc = jnp.zeros((LANES,), jnp.float32)
        for l in range(LANES):
            acc = acc + plsc.load_gather(part_v, [idx0 + l])
        red_v[pl.ds(g * LANES, LANES)] = acc
        return carry

    lax.fori_loop(0, 2 * PPW // LANES, red_body, 0)

    pltpu.sync_copy(
        red_v.at[pl.ds(0, PPW)],
        out_hbm.at[0, pl.ds(pbase, PPW)],
    )
    pltpu.sync_copy(
        red_v.at[pl.ds(PPW, PPW)],
        out_hbm.at[1, pl.ds(pbase, PPW)],
    )


def kernel(z_history, decay_rates, indices_i, indices_j):
    zT = jnp.swapaxes(z_history, 1, 2).reshape(B * D, T)
    gidx = jnp.stack(
        [indices_i, indices_j, indices_i + D, indices_j + D], axis=1
    ).reshape(-1).astype(jnp.int32)
    lw, qw = _weights_tc(decay_rates.reshape(P, 1))
    lw = lw.reshape(P * LANES)
    qw = qw.reshape(P * LANES)
    return _sync_sc(zT, gidx, lw, qw)
